# Initial kernel scaffold; baseline (speedup 1.0000x reference)
import jax
import jax.numpy as jnp
from jax.experimental import pallas as pl


def kernel(word_table, cap_table, postag_table, pos1_table, pos2_table, word_inputs, feature_inputs_0, feature_inputs_1, word_seq_lengths, position1_inputs, position2_inputs):
    raise NotImplementedError("write your pallas kernel here")


# speedup vs baseline: 2.1318x; 2.1318x over previous
import functools

import jax
import jax.numpy as jnp
from jax import lax
from jax.experimental import pallas as pl
from jax.experimental.pallas import tpu as pltpu
from jax.experimental.pallas import tpu_sc as plsc

B, L = 1024, 200
D_WORD, D_CAP, D_POS, D_PPOS = 128, 16, 32, 32
D_OUT = D_WORD + D_CAP + D_POS + D_PPOS + D_PPOS
N_TOK = B * L
CHUNK = 128
NW = 32
TOK_PER_W = N_TOK // NW
N_CHUNKS = TOK_PER_W // CHUNK

_SEGS = (
    (0, 0, 128),
    (1, 0, 144),
    (1, 16, 160),
    (2, 0, 176),
    (2, 16, 192),
    (3, 0, 208),
    (3, 16, 224),
)


def _wordrep_sc(wt, ct, tt, p1t, p2t, wi, ci, ti, p1i, p2i):
    mesh = plsc.VectorSubcoreMesh(core_axis_name="c", subcore_axis_name="s")

    @functools.partial(
        pl.kernel,
        mesh=mesh,
        out_type=jax.ShapeDtypeStruct((N_TOK, D_OUT), jnp.float32),
        scratch_types=[
            pltpu.VMEM((CHUNK,), jnp.int32),
            pltpu.VMEM((CHUNK,), jnp.int32),
            pltpu.VMEM((CHUNK,), jnp.int32),
            pltpu.VMEM((CHUNK,), jnp.int32),
            pltpu.VMEM((CHUNK,), jnp.int32),
            pltpu.VMEM((CHUNK, 128), jnp.float32),
            pltpu.VMEM((CHUNK, 128), jnp.float32),
            pltpu.VMEM((CHUNK, 128), jnp.float32),
            pltpu.VMEM((CHUNK, 128), jnp.float32),
            pltpu.VMEM((CHUNK, D_OUT), jnp.float32),
            pltpu.SemaphoreType.DMA,
        ],
    )
    def k(wt_h, ct_h, tt_h, p1t_h, p2t_h, wi_h, ci_h, ti_h, p1i_h, p2i_h,
          out_h, wi_v, ci_v, ti_v, p1i_v, p2i_v, cap_b, pt_b, p1_b, p2_b,
          stage, sem):
        wid = lax.axis_index("s") * 2 + lax.axis_index("c")
        tok0_w = wid * TOK_PER_W
        bufs = (cap_b, pt_b, p1_b, p2_b)

        def body(ch, carry):
            tok0 = tok0_w + ch * CHUNK
            pltpu.sync_copy(wi_h.at[pl.ds(tok0, CHUNK)], wi_v)
            pltpu.sync_copy(ci_h.at[pl.ds(tok0, CHUNK)], ci_v)
            pltpu.sync_copy(ti_h.at[pl.ds(tok0, CHUNK)], ti_v)
            pltpu.sync_copy(p1i_h.at[pl.ds(tok0, CHUNK)], p1i_v)
            pltpu.sync_copy(p2i_h.at[pl.ds(tok0, CHUNK)], p2i_v)
            copies = [
                pltpu.async_copy(
                    wt_h.at[wi_v], stage.at[:, pl.ds(0, D_WORD)], sem),
                pltpu.async_copy(ct_h.at[ci_v], cap_b, sem),
                pltpu.async_copy(tt_h.at[ti_v], pt_b, sem),
                pltpu.async_copy(p1t_h.at[p1i_v], p1_b, sem),
                pltpu.async_copy(p2t_h.at[p2i_v], p2_b, sem),
            ]
            for c in copies:
                c.wait()

            def fill(i, c2):
                for slot, soff, doff in _SEGS:
                    stage[i, pl.ds(doff, 16)] = bufs[slot][i, pl.ds(soff, 16)]
                return c2

            lax.fori_loop(0, CHUNK, fill, 0)
            pltpu.sync_copy(stage, out_h.at[pl.ds(tok0, CHUNK)])
            return carry

        lax.fori_loop(0, N_CHUNKS, body, 0)

    return k(wt, ct, tt, p1t, p2t, wi, ci, ti, p1i, p2i)


def _pad128(t):
    return jnp.pad(t, ((0, 0), (0, 128 - t.shape[1])))


def kernel(word_table, cap_table, postag_table, pos1_table, pos2_table,
           word_inputs, feature_inputs_0, feature_inputs_1, word_seq_lengths,
           position1_inputs, position2_inputs):
    del word_seq_lengths
    out = _wordrep_sc(
        word_table, _pad128(cap_table), _pad128(postag_table),
        _pad128(pos1_table), _pad128(pos2_table),
        word_inputs.reshape(-1), feature_inputs_0.reshape(-1),
        feature_inputs_1.reshape(-1), position1_inputs.reshape(-1),
        position2_inputs.reshape(-1))
    return out.reshape(B, L, D_OUT)


# speedup vs baseline: 3.0390x; 1.4256x over previous
import functools

import jax
import jax.numpy as jnp
from jax import lax
from jax.experimental import pallas as pl
from jax.experimental.pallas import tpu as pltpu
from jax.experimental.pallas import tpu_sc as plsc

B, L = 1024, 200
D_WORD = 128
D_OUT = 240
N_TOK = B * L
CHUNK = 128
NW = 32
TOK_PER_W = N_TOK // NW
N_CHUNKS = TOK_PER_W // CHUNK


def _splat(x):
    return lax.broadcast_in_dim(x, (16,), ())


def _wordrep_sc(wt, ct, tt, p1t, p2t, wi, ci, ti, p1i, p2i):
    mesh = plsc.VectorSubcoreMesh(core_axis_name="c", subcore_axis_name="s")

    @functools.partial(
        pl.kernel,
        mesh=mesh,
        out_type=jax.ShapeDtypeStruct((N_TOK, D_OUT), jnp.float32),
        compiler_params=pltpu.CompilerParams(needs_layout_passes=False),
        scratch_types=[
            pltpu.VMEM((CHUNK,), jnp.int32),
            pltpu.VMEM((CHUNK,), jnp.int32),
            pltpu.VMEM((CHUNK,), jnp.int32),
            pltpu.VMEM((CHUNK,), jnp.int32),
            pltpu.VMEM((CHUNK,), jnp.int32),
            pltpu.VMEM((1, 128), jnp.float32),
            pltpu.VMEM((16, 128), jnp.float32),
            pltpu.VMEM((250, 128), jnp.float32),
            pltpu.VMEM((250, 128), jnp.float32),
            pltpu.VMEM((CHUNK, D_OUT), jnp.float32),
            pltpu.SemaphoreType.DMA,
        ],
    )
    def k(wt_h, ct_h, tt_h, p1t_h, p2t_h, wi_h, ci_h, ti_h, p1i_h, p2i_h,
          out_h, wi_v, ci_v, ti_v, p1i_v, p2i_v,
          cap_v, pt_v, p1_v, p2_v, stage, sem):
        wid = lax.axis_index("s") * 2 + lax.axis_index("c")
        tok0_w = wid * TOK_PER_W
        pltpu.sync_copy(ct_h, cap_v)
        pltpu.sync_copy(tt_h, pt_v)
        pltpu.sync_copy(p1t_h, p1_v)
        pltpu.sync_copy(p2t_h, p2_v)
        iota = lax.iota(jnp.int32, 16)

        def body(ch, carry):
            tok0 = tok0_w + ch * CHUNK
            pltpu.sync_copy(wi_h.at[pl.ds(tok0, CHUNK)], wi_v)
            pltpu.sync_copy(ci_h.at[pl.ds(tok0, CHUNK)], ci_v)
            pltpu.sync_copy(ti_h.at[pl.ds(tok0, CHUNK)], ti_v)
            pltpu.sync_copy(p1i_h.at[pl.ds(tok0, CHUNK)], p1i_v)
            pltpu.sync_copy(p2i_h.at[pl.ds(tok0, CHUNK)], p2i_v)
            wcopy = pltpu.async_copy(
                wt_h.at[wi_v], stage.at[:, pl.ds(0, D_WORD)], sem)

            def fill(k, c2):
                tok = k * 16 + iota
                sl = pl.ds(k * 16, 16)
                zeros = _splat(0)
                ccol = ci_v[sl] * 16
                t = ti_v[sl]
                trow, tcol = t >> 2, (t & 3) * 32
                q1 = p1i_v[sl]
                r1, b1 = q1 >> 2, (q1 & 3) * 32
                q2 = p2i_v[sl]
                r2, b2 = q2 >> 2, (q2 & 3) * 32
                feats = (
                    (cap_v, zeros, ccol, 128, 16),
                    (pt_v, trow, tcol, 144, 32),
                    (p1_v, r1, b1, 176, 32),
                    (p2_v, r2, b2, 208, 32),
                )
                for ref, row, colb, doff, n in feats:
                    for j in range(n):
                        vals = plsc.load_gather(ref, [row, colb + j])
                        plsc.store_scatter(
                            stage, [tok, _splat(doff + j)], vals)
                return c2

            lax.fori_loop(0, CHUNK // 16, fill, 0)
            wcopy.wait()
            pltpu.sync_copy(stage, out_h.at[pl.ds(tok0, CHUNK)])
            return carry

        lax.fori_loop(0, N_CHUNKS, body, 0)

    return k(wt, ct, tt, p1t, p2t, wi, ci, ti, p1i, p2i)


def kernel(word_table, cap_table, postag_table, pos1_table, pos2_table,
           word_inputs, feature_inputs_0, feature_inputs_1, word_seq_lengths,
           position1_inputs, position2_inputs):
    del word_seq_lengths
    out = _wordrep_sc(
        word_table, cap_table.reshape(1, 128), postag_table.reshape(16, 128),
        pos1_table.reshape(250, 128), pos2_table.reshape(250, 128),
        word_inputs.reshape(-1), feature_inputs_0.reshape(-1),
        feature_inputs_1.reshape(-1), position1_inputs.reshape(-1),
        position2_inputs.reshape(-1))
    return out.reshape(B, L, D_OUT)


# speedup vs baseline: 3.3455x; 1.1009x over previous
import functools

import jax
import jax.numpy as jnp
from jax import lax
from jax.experimental import pallas as pl
from jax.experimental.pallas import tpu as pltpu
from jax.experimental.pallas import tpu_sc as plsc

B, L = 1024, 200
D_WORD = 128
D_OUT = 240
N_TOK = B * L
CHUNK = 64
NW = 32
TOK_PER_W = N_TOK // NW
N_CHUNKS = TOK_PER_W // CHUNK
NBUF = 3
N_GROUPS = (N_CHUNKS + NBUF - 1) // NBUF


def _splat(x):
    return lax.broadcast_in_dim(x, (16,), ())


def _wordrep_sc(wt, ct, tt, p1t, p2t, idx5):
    mesh = plsc.VectorSubcoreMesh(core_axis_name="c", subcore_axis_name="s")

    @functools.partial(
        pl.kernel,
        mesh=mesh,
        out_type=jax.ShapeDtypeStruct((N_TOK, D_OUT), jnp.float32),
        compiler_params=pltpu.CompilerParams(needs_layout_passes=False),
        scratch_types=[
            [pltpu.VMEM((5, CHUNK), jnp.int32) for _ in range(NBUF)],
            [pltpu.VMEM((CHUNK, D_OUT), jnp.float32) for _ in range(NBUF)],
            pltpu.VMEM((1, 128), jnp.float32),
            pltpu.VMEM((16, 128), jnp.float32),
            pltpu.VMEM((250, 128), jnp.float32),
            pltpu.VMEM((250, 128), jnp.float32),
            [pltpu.SemaphoreType.DMA for _ in range(NBUF)],
            [pltpu.SemaphoreType.DMA for _ in range(NBUF)],
            [pltpu.SemaphoreType.DMA for _ in range(NBUF)],
        ],
    )
    def k(wt_h, ct_h, tt_h, p1t_h, p2t_h, idx5_h, out_h,
          idx_v, stage, cap_v, pt_v, p1_v, p2_v, semi, semg, sems):
        wid = lax.axis_index("s") * 2 + lax.axis_index("c")
        chunk0 = wid * N_CHUNKS
        pltpu.sync_copy(ct_h, cap_v)
        pltpu.sync_copy(tt_h, pt_v)
        pltpu.sync_copy(p1t_h, p1_v)
        pltpu.sync_copy(p2t_h, p2_v)
        iota = lax.iota(jnp.int32, 16)

        def issue_gather(b):
            return pltpu.async_copy(
                wt_h.at[idx_v[b].at[0]],
                stage[b].at[:, pl.ds(0, D_WORD)], semg[b])

        def wait_gather(b):
            pltpu.make_async_copy(
                wt_h.at[pl.ds(0, CHUNK)],
                stage[b].at[:, pl.ds(0, D_WORD)], semg[b]).wait()

        def wait_idx(b):
            pltpu.make_async_copy(idx5_h.at[0], idx_v[b], semi[b]).wait()

        def wait_store(b):
            pltpu.make_async_copy(
                stage[b], out_h.at[pl.ds(0, CHUNK)], sems[b]).wait()

        def fill(b):
            ref = idx_v[b]

            def fill_step(kk, c2):
                tok = kk * 16 + iota
                sl = pl.ds(kk * 16, 16)
                zeros = _splat(0)
                ccol = ref[1, sl] * 16
                t = ref[2, sl]
                trow, tcol = t >> 2, (t & 3) * 32
                q1 = ref[3, sl]
                r1, b1 = q1 >> 2, (q1 & 3) * 32
                q2 = ref[4, sl]
                r2, b2 = q2 >> 2, (q2 & 3) * 32
                feats = (
                    (cap_v, zeros, ccol, 128, 16),
                    (pt_v, trow, tcol, 144, 32),
                    (p1_v, r1, b1, 176, 32),
                    (p2_v, r2, b2, 208, 32),
                )
                for tab, row, colb, doff, n in feats:
                    for j in range(n):
                        vals = plsc.load_gather(tab, [row, colb + j])
                        plsc.store_scatter(
                            stage[b], [tok, _splat(doff + j)], vals)
                return c2

            lax.fori_loop(0, CHUNK // 16, fill_step, 0)

        for b in range(NBUF):
            pltpu.sync_copy(idx5_h.at[chunk0 + b], idx_v[b])
            issue_gather(b)

        def body(g, carry):
            for b in range(NBUF):
                c = g * NBUF + b

                @pl.when(c < N_CHUNKS)
                def _():
                    wait_gather(b)
                    fill(b)
                    pltpu.async_copy(
                        stage[b],
                        out_h.at[pl.ds((chunk0 + c) * CHUNK, CHUNK)], sems[b])

                @pl.when(c + NBUF < N_CHUNKS)
                def _():
                    pltpu.async_copy(
                        idx5_h.at[chunk0 + c + NBUF], idx_v[b], semi[b])

            for b in range(NBUF):
                c = g * NBUF + b

                @pl.when(c + NBUF < N_CHUNKS)
                def _():
                    wait_idx(b)
                    wait_store(b)
                    issue_gather(b)

            return carry

        lax.fori_loop(0, N_GROUPS, body, 0)
        for b in range(NBUF):
            wait_store(b)

    return k(wt, ct, tt, p1t, p2t, idx5)


def kernel(word_table, cap_table, postag_table, pos1_table, pos2_table,
           word_inputs, feature_inputs_0, feature_inputs_1, word_seq_lengths,
           position1_inputs, position2_inputs):
    del word_seq_lengths
    idx5 = jnp.stack(
        [word_inputs.reshape(-1, CHUNK), feature_inputs_0.reshape(-1, CHUNK),
         feature_inputs_1.reshape(-1, CHUNK),
         position1_inputs.reshape(-1, CHUNK),
         position2_inputs.reshape(-1, CHUNK)], axis=1)
    out = _wordrep_sc(
        word_table, cap_table.reshape(1, 128), postag_table.reshape(16, 128),
        pos1_table.reshape(250, 128), pos2_table.reshape(250, 128), idx5)
    return out.reshape(B, L, D_OUT)


# speedup vs baseline: 4.2649x; 1.2748x over previous
import functools

import jax
import jax.numpy as jnp
from jax import lax
from jax.experimental import pallas as pl
from jax.experimental.pallas import tpu as pltpu
from jax.experimental.pallas import tpu_sc as plsc

B, L = 1024, 200
D_WORD = 128
D_OUT = 240
N_TOK = B * L
CHUNK = 64
NW = 32
TOK_PER_W = N_TOK // NW
N_CHUNKS = TOK_PER_W // CHUNK
NBUF = 3
N_GROUPS = (N_CHUNKS + NBUF - 1) // NBUF


def _splat(x):
    return lax.broadcast_in_dim(x, (16,), ())


def _wordrep_sc(wt, ct, tt, p1t, p2t, idx5):
    mesh = plsc.VectorSubcoreMesh(core_axis_name="c", subcore_axis_name="s")

    @functools.partial(
        pl.kernel,
        mesh=mesh,
        out_type=jax.ShapeDtypeStruct((N_TOK, D_OUT), jnp.float32),
        compiler_params=pltpu.CompilerParams(needs_layout_passes=False),
        scratch_types=[
            [pltpu.VMEM((5, CHUNK), jnp.int32) for _ in range(NBUF)],
            [pltpu.VMEM((CHUNK, D_OUT), jnp.float32) for _ in range(NBUF)],
            pltpu.VMEM((1, 128), jnp.float32),
            pltpu.VMEM((16, 128), jnp.float32),
            pltpu.VMEM((250, 128), jnp.float32),
            pltpu.VMEM((250, 128), jnp.float32),
            [pltpu.SemaphoreType.DMA for _ in range(NBUF)],
            [pltpu.SemaphoreType.DMA for _ in range(NBUF)],
            [pltpu.SemaphoreType.DMA for _ in range(NBUF)],
        ],
    )
    def k(wt_h, ct_h, tt_h, p1t_h, p2t_h, idx5_h, out_h,
          idx_v, stage, cap_v, pt_v, p1_v, p2_v, semi, semg, sems):
        wid = lax.axis_index("s") * 2 + lax.axis_index("c")
        chunk0 = wid * N_CHUNKS
        pltpu.sync_copy(ct_h, cap_v)
        pltpu.sync_copy(tt_h, pt_v)
        pltpu.sync_copy(p1t_h, p1_v)
        pltpu.sync_copy(p2t_h, p2_v)
        iota = lax.iota(jnp.int32, 16)

        def issue_gather(b):
            return pltpu.async_copy(
                wt_h.at[idx_v[b].at[0]],
                stage[b].at[:, pl.ds(0, D_WORD)], semg[b])

        def wait_gather(b):
            pltpu.make_async_copy(
                wt_h.at[pl.ds(0, CHUNK)],
                stage[b].at[:, pl.ds(0, D_WORD)], semg[b]).wait()

        def wait_idx(b):
            pltpu.make_async_copy(idx5_h.at[0], idx_v[b], semi[b]).wait()

        def wait_store(b):
            pltpu.make_async_copy(
                stage[b], out_h.at[pl.ds(0, CHUNK)], sems[b]).wait()

        def fill(b):
            ref = idx_v[b]

            def fill_step(kk, c2):
                tok = kk * 16 + iota
                sl = pl.ds(kk * 16, 16)
                zeros = _splat(0)
                ccol = ref[1, sl] * 16
                t = ref[2, sl]
                trow, tcol = t >> 2, (t & 3) * 32
                q1 = ref[3, sl]
                r1, b1 = q1 >> 2, (q1 & 3) * 32
                q2 = ref[4, sl]
                r2, b2 = q2 >> 2, (q2 & 3) * 32
                pairs = []
                for tab, row, colb, doff, n in (
                        (cap_v, zeros, ccol, 128, 16),
                        (pt_v, trow, tcol, 144, 32),
                        (p1_v, r1, b1, 176, 32),
                        (p2_v, r2, b2, 208, 32)):
                    pairs.extend((tab, row, colb, doff + j, j) for j in range(n))
                for base in range(0, len(pairs), 16):
                    batch = pairs[base:base + 16]
                    vals = [plsc.load_gather(tab, [row, colb + j])
                            for tab, row, colb, _, j in batch]
                    for v, (_, _, _, dcol, _) in zip(vals, batch):
                        plsc.store_scatter(stage[b], [tok, _splat(dcol)], v)
                return c2

            lax.fori_loop(0, CHUNK // 16, fill_step, 0)

        for b in range(NBUF):
            pltpu.sync_copy(idx5_h.at[chunk0 + b], idx_v[b])
            issue_gather(b)

        def body(g, carry):
            for b in range(NBUF):
                c = g * NBUF + b

                @pl.when(c < N_CHUNKS)
                def _():
                    wait_gather(b)
                    fill(b)
                    pltpu.async_copy(
                        stage[b],
                        out_h.at[pl.ds((chunk0 + c) * CHUNK, CHUNK)], sems[b])

                @pl.when(c + NBUF < N_CHUNKS)
                def _():
                    pltpu.async_copy(
                        idx5_h.at[chunk0 + c + NBUF], idx_v[b], semi[b])

            for b in range(NBUF):
                c = g * NBUF + b

                @pl.when(c + NBUF < N_CHUNKS)
                def _():
                    wait_idx(b)
                    wait_store(b)
                    issue_gather(b)

            return carry

        lax.fori_loop(0, N_GROUPS, body, 0)
        for b in range(NBUF):
            wait_store(b)

    return k(wt, ct, tt, p1t, p2t, idx5)


def kernel(word_table, cap_table, postag_table, pos1_table, pos2_table,
           word_inputs, feature_inputs_0, feature_inputs_1, word_seq_lengths,
           position1_inputs, position2_inputs):
    del word_seq_lengths
    idx5 = jnp.stack(
        [word_inputs.reshape(-1, CHUNK), feature_inputs_0.reshape(-1, CHUNK),
         feature_inputs_1.reshape(-1, CHUNK),
         position1_inputs.reshape(-1, CHUNK),
         position2_inputs.reshape(-1, CHUNK)], axis=1)
    out = _wordrep_sc(
        word_table, cap_table.reshape(1, 128), postag_table.reshape(16, 128),
        pos1_table.reshape(250, 128), pos2_table.reshape(250, 128), idx5)
    return out.reshape(B, L, D_OUT)


# speedup vs baseline: 4.4619x; 1.0462x over previous
import functools

import jax
import jax.numpy as jnp
from jax import lax
from jax.experimental import pallas as pl
from jax.experimental.pallas import tpu as pltpu
from jax.experimental.pallas import tpu_sc as plsc

B, L = 1024, 200
D_WORD = 128
D_OUT = 240
N_TOK = B * L
CHUNK = 64
NW = 32
TOK_PER_W = N_TOK // NW
N_CHUNKS = TOK_PER_W // CHUNK
NBUF = 3
N_GROUPS = (N_CHUNKS + NBUF - 1) // NBUF


def _splat(x):
    return lax.broadcast_in_dim(x, (16,), ())


def _wordrep_sc(wt, ct, tt, p1t, p2t, idx5):
    mesh = plsc.VectorSubcoreMesh(core_axis_name="c", subcore_axis_name="s")

    @functools.partial(
        pl.kernel,
        mesh=mesh,
        out_type=jax.ShapeDtypeStruct((N_TOK, D_OUT), jnp.float32),
        compiler_params=pltpu.CompilerParams(needs_layout_passes=False),
        scratch_types=[
            [pltpu.VMEM((5, CHUNK), jnp.int32) for _ in range(NBUF)],
            [pltpu.VMEM((CHUNK, D_OUT), jnp.float32) for _ in range(NBUF)],
            pltpu.VMEM((1, 128), jnp.float32),
            pltpu.VMEM((16, 128), jnp.float32),
            pltpu.VMEM((250, 128), jnp.float32),
            pltpu.VMEM((250, 128), jnp.float32),
            [pltpu.SemaphoreType.DMA for _ in range(NBUF)],
            [pltpu.SemaphoreType.DMA for _ in range(NBUF)],
            [pltpu.SemaphoreType.DMA for _ in range(NBUF)],
        ],
    )
    def k(wt_h, ct_h, tt_h, p1t_h, p2t_h, idx5_h, out_h,
          idx_v, stage, cap_v, pt_v, p1_v, p2_v, semi, semg, sems):
        wid = lax.axis_index("s") * 2 + lax.axis_index("c")
        chunk0 = wid * N_CHUNKS
        pltpu.sync_copy(ct_h, cap_v)
        pltpu.sync_copy(tt_h, pt_v)
        pltpu.sync_copy(p1t_h, p1_v)
        pltpu.sync_copy(p2t_h, p2_v)
        iota = lax.iota(jnp.int32, 16)

        def issue_gather(b):
            return pltpu.async_copy(
                wt_h.at[idx_v[b].at[0]],
                stage[b].at[:, pl.ds(0, D_WORD)], semg[b])

        def wait_gather(b):
            pltpu.make_async_copy(
                wt_h.at[pl.ds(0, CHUNK)],
                stage[b].at[:, pl.ds(0, D_WORD)], semg[b]).wait()

        def wait_idx(b):
            pltpu.make_async_copy(idx5_h.at[0], idx_v[b], semi[b]).wait()

        def wait_store(b):
            pltpu.make_async_copy(
                stage[b], out_h.at[pl.ds(0, CHUNK)], sems[b]).wait()

        def fill(b):
            ref = idx_v[b]

            def fill_step(kk, c2):
                tok = kk * 16 + iota
                sl = pl.ds(kk * 16, 16)
                zeros = _splat(0)
                ccol = ref[1, sl] * 16
                t = ref[2, sl]
                trow, tcol = t >> 2, (t & 3) * 32
                q1 = ref[3, sl]
                r1, b1 = q1 >> 2, (q1 & 3) * 32
                q2 = ref[4, sl]
                r2, b2 = q2 >> 2, (q2 & 3) * 32
                pairs = []
                for tab, row, colb, doff, n in (
                        (cap_v, zeros, ccol, 128, 16),
                        (pt_v, trow, tcol, 144, 32),
                        (p1_v, r1, b1, 176, 32),
                        (p2_v, r2, b2, 208, 32)):
                    pairs.extend((tab, row, colb, doff + j, j) for j in range(n))
                for base in range(0, len(pairs), 16):
                    batch = pairs[base:base + 16]
                    vals = [plsc.load_gather(tab, [row, colb + j])
                            for tab, row, colb, _, j in batch]
                    for v, (_, _, _, dcol, _) in zip(vals, batch):
                        plsc.store_scatter(stage[b], [tok, _splat(dcol)], v)
                return c2

            lax.fori_loop(0, CHUNK // 16, fill_step, 0)

        for b in range(NBUF):
            pltpu.sync_copy(idx5_h.at[chunk0 + b], idx_v[b])
            issue_gather(b)

        def body(g, carry):
            for b in range(NBUF):
                c = g * NBUF + b

                @pl.when(c < N_CHUNKS)
                def _():
                    wait_gather(b)
                    fill(b)
                    pltpu.async_copy(
                        stage[b],
                        out_h.at[pl.ds((chunk0 + c) * CHUNK, CHUNK)], sems[b])

                @pl.when(c + NBUF < N_CHUNKS)
                def _():
                    pltpu.async_copy(
                        idx5_h.at[chunk0 + c + NBUF], idx_v[b], semi[b])

                cc = c + 2
                bb = (b + 2) % NBUF

                @pl.when((cc >= NBUF) & (cc < N_CHUNKS))
                def _():
                    wait_idx(bb)
                    wait_store(bb)
                    issue_gather(bb)

            return carry

        lax.fori_loop(0, N_GROUPS, body, 0)
        for b in range(NBUF):
            wait_store(b)

    return k(wt, ct, tt, p1t, p2t, idx5)


def kernel(word_table, cap_table, postag_table, pos1_table, pos2_table,
           word_inputs, feature_inputs_0, feature_inputs_1, word_seq_lengths,
           position1_inputs, position2_inputs):
    del word_seq_lengths
    idx5 = jnp.stack(
        [word_inputs.reshape(-1, CHUNK), feature_inputs_0.reshape(-1, CHUNK),
         feature_inputs_1.reshape(-1, CHUNK),
         position1_inputs.reshape(-1, CHUNK),
         position2_inputs.reshape(-1, CHUNK)], axis=1)
    out = _wordrep_sc(
        word_table, cap_table.reshape(1, 128), postag_table.reshape(16, 128),
        pos1_table.reshape(250, 128), pos2_table.reshape(250, 128), idx5)
    return out.reshape(B, L, D_OUT)


# speedup vs baseline: 8.5247x; 1.9105x over previous
import functools

import jax
import jax.numpy as jnp
from jax import lax
from jax.experimental import pallas as pl
from jax.experimental.pallas import tpu as pltpu
from jax.experimental.pallas import tpu_sc as plsc

B, L = 1024, 200
D_WORD = 128
D_OUT = 240
N_TOK = B * L
CHUNK = 64
NW = 32
TOK_PER_W = N_TOK // NW
N_CHUNKS = TOK_PER_W // CHUNK
NBUF = 3
N_GROUPS = (N_CHUNKS + NBUF - 1) // NBUF


def _splat(x):
    return lax.broadcast_in_dim(x, (16,), ())


def _wordrep_sc(wt, ct, tt, p1t, p2t, idx5):
    mesh = plsc.VectorSubcoreMesh(core_axis_name="c", subcore_axis_name="s")

    @functools.partial(
        pl.kernel,
        mesh=mesh,
        out_type=jax.ShapeDtypeStruct((N_TOK, D_OUT), jnp.float32),
        compiler_params=pltpu.CompilerParams(needs_layout_passes=False),
        scratch_types=[
            [pltpu.VMEM((5, CHUNK), jnp.int32) for _ in range(NBUF)],
            [pltpu.VMEM((CHUNK, D_OUT), jnp.float32) for _ in range(NBUF)],
            pltpu.VMEM((1, 128), jnp.float32),
            pltpu.VMEM((16, 128), jnp.float32),
            pltpu.VMEM((250, 128), jnp.float32),
            pltpu.VMEM((250, 128), jnp.float32),
            [pltpu.SemaphoreType.DMA for _ in range(NBUF)],
            [pltpu.SemaphoreType.DMA for _ in range(NBUF)],
            [pltpu.SemaphoreType.DMA for _ in range(NBUF)],
        ],
    )
    def k(wt_h, ct_h, tt_h, p1t_h, p2t_h, idx5_h, out_h,
          idx_v, stage, cap_v, pt_v, p1_v, p2_v, semi, semg, sems):
        wid = lax.axis_index("s") * 2 + lax.axis_index("c")
        chunk0 = wid * N_CHUNKS
        pltpu.sync_copy(ct_h, cap_v)
        pltpu.sync_copy(tt_h, pt_v)
        pltpu.sync_copy(p1t_h, p1_v)
        pltpu.sync_copy(p2t_h, p2_v)
        iota = lax.iota(jnp.int32, 16)

        def issue_gather(b):
            return pltpu.async_copy(
                wt_h.at[idx_v[b].at[0]],
                stage[b].at[:, pl.ds(0, D_WORD)], semg[b])

        def wait_gather(b):
            pltpu.make_async_copy(
                wt_h.at[pl.ds(0, CHUNK)],
                stage[b].at[:, pl.ds(0, D_WORD)], semg[b]).wait()

        def wait_idx(b):
            pltpu.make_async_copy(idx5_h.at[0], idx_v[b], semi[b]).wait()

        def wait_store(b):
            pltpu.make_async_copy(
                stage[b], out_h.at[pl.ds(0, CHUNK)], sems[b]).wait()

        def fill(b):
            ref = idx_v[b]

            def fill_step(kk, c2):
                tok = kk * 16 + iota
                sl = pl.ds(kk * 16, 16)
                zeros = _splat(0)
                ccol = ref[1, sl] * 16
                t = ref[2, sl]
                trow, tcol = t >> 2, (t & 3) * 32
                q1 = ref[3, sl]
                r1, b1 = q1 >> 2, (q1 & 3) * 32
                q2 = ref[4, sl]
                r2, b2 = q2 >> 2, (q2 & 3) * 32
                subsegs = (
                    (cap_v, zeros, ccol, 128),
                    (pt_v, trow, tcol, 144),
                    (pt_v, trow, tcol + 16, 160),
                    (p1_v, r1, b1, 176),
                    (p1_v, r1, b1 + 16, 192),
                    (p2_v, r2, b2, 208),
                    (p2_v, r2, b2 + 16, 224),
                )
                def jstep(j, c3):
                    rot = (iota + j) & 15
                    for tab, row, cb, dcol in subsegs:
                        v = plsc.load_gather(tab, [row, cb + rot])
                        plsc.store_scatter(stage[b], [tok, rot + dcol], v)
                    return c3

                lax.fori_loop(0, 16, jstep, 0)
                return c2

            lax.fori_loop(0, CHUNK // 16, fill_step, 0)

        for b in range(NBUF):
            pltpu.sync_copy(idx5_h.at[chunk0 + b], idx_v[b])
            issue_gather(b)

        def body(g, carry):
            for b in range(NBUF):
                c = g * NBUF + b

                @pl.when(c < N_CHUNKS)
                def _():
                    wait_gather(b)
                    fill(b)
                    pltpu.async_copy(
                        stage[b],
                        out_h.at[pl.ds((chunk0 + c) * CHUNK, CHUNK)], sems[b])

                @pl.when(c + NBUF < N_CHUNKS)
                def _():
                    pltpu.async_copy(
                        idx5_h.at[chunk0 + c + NBUF], idx_v[b], semi[b])

                cc = c + 2
                bb = (b + 2) % NBUF

                @pl.when((cc >= NBUF) & (cc < N_CHUNKS))
                def _():
                    wait_idx(bb)
                    wait_store(bb)
                    issue_gather(bb)

            return carry

        lax.fori_loop(0, N_GROUPS, body, 0)
        for b in range(NBUF):
            wait_store(b)

    return k(wt, ct, tt, p1t, p2t, idx5)


def kernel(word_table, cap_table, postag_table, pos1_table, pos2_table,
           word_inputs, feature_inputs_0, feature_inputs_1, word_seq_lengths,
           position1_inputs, position2_inputs):
    del word_seq_lengths
    idx5 = jnp.stack(
        [word_inputs.reshape(-1, CHUNK), feature_inputs_0.reshape(-1, CHUNK),
         feature_inputs_1.reshape(-1, CHUNK),
         position1_inputs.reshape(-1, CHUNK),
         position2_inputs.reshape(-1, CHUNK)], axis=1)
    out = _wordrep_sc(
        word_table, cap_table.reshape(1, 128), postag_table.reshape(16, 128),
        pos1_table.reshape(250, 128), pos2_table.reshape(250, 128), idx5)
    return out.reshape(B, L, D_OUT)
